# Initial kernel scaffold; baseline (speedup 1.0000x reference)
import functools

import jax
import jax.numpy as jnp
from jax import lax
from jax.experimental import pallas as pl
from jax.experimental.pallas import tpu as pltpu
from jax.experimental.pallas import tpu_sc as plsc

NC = 2
NS = 16
NW = NC * NS
LANES = 16
C = 128

N = 10000
D = 128
E = 320000

N_CH = -(-E // (NW * C))
E_PAD = NW * C * N_CH
DEG_PAD = 10240
DEG_PER_TILE = DEG_PAD // NS
ACC_ROWS = 10016
ROWS_PER_TILE = ACC_ROWS // NS


def _zero_vmem_2d(buf, rows):
    def body(r, _):
        z = jnp.zeros((LANES,), jnp.float32)
        for i in range(D // LANES):
            buf[r, pl.ds(i * LANES, LANES)] = z
        return 0
    lax.fori_loop(0, rows, body, 0)


def _zero_vmem_1d(buf, n):
    def body(r, _):
        buf[pl.ds(r * LANES, LANES)] = jnp.zeros((LANES,), jnp.float32)
        return 0
    lax.fori_loop(0, n // LANES, body, 0)


def _deg_body(col_hbm, out_hbm, col_v, ones_v, zrow_v, acc_s):
    c = lax.axis_index("c")
    s = lax.axis_index("s")
    pltpu.sync_copy(col_hbm.at[c, s], col_v)
    _zero_vmem_1d(zrow_v, DEG_PER_TILE)
    def wone(i, _):
        ones_v[pl.ds(i * LANES, LANES)] = jnp.full((LANES,), 1.0, jnp.float32)
        return 0
    lax.fori_loop(0, C // LANES, wone, 0)
    pltpu.sync_copy(zrow_v, acc_s.at[pl.ds(s * DEG_PER_TILE, DEG_PER_TILE)])
    plsc.subcore_barrier()
    def chunk(j, _):
        pltpu.sync_copy(ones_v, acc_s.at[col_v.at[j]], add=True)
        return 0
    lax.fori_loop(0, N_CH, chunk, 0)
    plsc.subcore_barrier()
    sl = pl.ds(s * DEG_PER_TILE, DEG_PER_TILE)
    pltpu.sync_copy(acc_s.at[sl], zrow_v)
    pltpu.sync_copy(zrow_v, out_hbm.at[c, sl])


def _hop_body(h_hbm, row_hbm, col_hbm, out_hbm,
              row_v, col_v, buf0, buf1, acc_s, gsem, ssem):
    c = lax.axis_index("c")
    s = lax.axis_index("s")
    pltpu.sync_copy(row_hbm.at[c, s], row_v)
    pltpu.sync_copy(col_hbm.at[c, s], col_v)
    _zero_vmem_2d(buf0, C)
    base = s * ROWS_PER_TILE
    for k in range(4):
        pltpu.sync_copy(buf0, acc_s.at[pl.ds(base + k * C, C)])
    pltpu.sync_copy(buf0.at[pl.ds(0, ROWS_PER_TILE - 4 * C)],
                    acc_s.at[pl.ds(base + 4 * C, ROWS_PER_TILE - 4 * C)])
    plsc.subcore_barrier()

    half = N_CH // 2

    def gather(j, buf):
        return pltpu.make_async_copy(h_hbm.at[row_v.at[j]], buf, gsem)

    def scat(j, buf):
        return pltpu.make_async_copy(buf, acc_s.at[col_v.at[j]], ssem)

    gather(0, buf0).start()

    def pair(p, _):
        j0 = 2 * p
        gather(j0, buf0).wait()

        @pl.when(p > 0)
        def _():
            scat(j0, buf1).wait()
        gather(j0 + 1, buf1).start()
        scat(j0, buf0).start(add=True)
        gather(j0 + 1, buf1).wait()

        @pl.when(p < half - 1)
        def _():
            gather(j0 + 2, buf0).start()
        scat(j0, buf0).wait()
        scat(j0 + 1, buf1).start(add=True)
        return 0

    lax.fori_loop(0, half, pair, 0)
    scat(0, buf1).wait()
    plsc.subcore_barrier()

    for k in range(4):
        sl = pl.ds(base + k * C, C)
        pltpu.sync_copy(acc_s.at[sl], buf0)
        pltpu.sync_copy(buf0, out_hbm.at[c, sl, :])
    rem = ROWS_PER_TILE - 4 * C
    sl = pl.ds(base + 4 * C, rem)
    pltpu.sync_copy(acc_s.at[sl], buf0.at[pl.ds(0, rem)])
    pltpu.sync_copy(buf0.at[pl.ds(0, rem)], out_hbm.at[c, sl, :])


_MESH = plsc.VectorSubcoreMesh(core_axis_name="c", subcore_axis_name="s",
                               num_cores=NC, num_subcores=NS)

_deg_call = functools.partial(
    pl.kernel, _deg_body,
    out_type=jax.ShapeDtypeStruct((NC, DEG_PAD), jnp.float32),
    mesh=_MESH,
    scratch_types=[
        pltpu.VMEM((N_CH, C), jnp.int32),
        pltpu.VMEM((C,), jnp.float32),
        pltpu.VMEM((DEG_PER_TILE,), jnp.float32),
        pltpu.VMEM_SHARED((DEG_PAD,), jnp.float32),
    ],
)()

_hop_call = functools.partial(
    pl.kernel, _hop_body,
    out_type=jax.ShapeDtypeStruct((NC, ACC_ROWS, D), jnp.float32),
    mesh=_MESH,
    scratch_types=[
        pltpu.VMEM((N_CH, C), jnp.int32),
        pltpu.VMEM((N_CH, C), jnp.int32),
        pltpu.VMEM((C, D), jnp.float32),
        pltpu.VMEM((C, D), jnp.float32),
        pltpu.VMEM_SHARED((ACC_ROWS, D), jnp.float32),
        pltpu.SemaphoreType.DMA,
        pltpu.SemaphoreType.DMA,
    ],
)()


_RB = 2000
_GRID = N // _RB


def _prep_body(p0, p1, x_ref, xs, dis, dis2):
    deg = p0[...] + p1[...]
    d = lax.rsqrt(jnp.maximum(deg, 1e-12))
    dd = jnp.where(deg > 0, d, 0.0)
    dis[...] = dd
    dis2[...] = dd * dd
    xs[...] = x_ref[...] * dd


def _prep_tc(p0, p1, x):
    return pl.pallas_call(
        _prep_body,
        grid=(_GRID,),
        in_specs=[
            pl.BlockSpec((_RB, 1), lambda i: (i, 0)),
            pl.BlockSpec((_RB, 1), lambda i: (i, 0)),
            pl.BlockSpec((_RB, D), lambda i: (i, 0)),
        ],
        out_specs=[
            pl.BlockSpec((_RB, D), lambda i: (i, 0)),
            pl.BlockSpec((_RB, 1), lambda i: (i, 0)),
            pl.BlockSpec((_RB, 1), lambda i: (i, 0)),
        ],
        out_shape=[
            jax.ShapeDtypeStruct((N, D), jnp.float32),
            jax.ShapeDtypeStruct((N, 1), jnp.float32),
            jax.ShapeDtypeStruct((N, 1), jnp.float32),
        ],
    )(p0, p1, x)


def _mid_body(a0, a1, dis2, o):
    o[...] = (a0[...] + a1[...]) * dis2[...]


def _mid_tc(a0, a1, dis2):
    return pl.pallas_call(
        _mid_body,
        grid=(_GRID,),
        in_specs=[
            pl.BlockSpec((_RB, D), lambda i: (i, 0)),
            pl.BlockSpec((_RB, D), lambda i: (i, 0)),
            pl.BlockSpec((_RB, 1), lambda i: (i, 0)),
        ],
        out_specs=pl.BlockSpec((_RB, D), lambda i: (i, 0)),
        out_shape=jax.ShapeDtypeStruct((N, D), jnp.float32),
    )(a0, a1, dis2)


def _fin_body(a0, a1, dis, w_ref, b_ref, o):
    h = (a0[...] + a1[...]) * dis[...]
    o[...] = lax.dot_general(h, w_ref[...], (((1,), (1,)), ((), ())),
                             preferred_element_type=jnp.float32) + b_ref[...]


def _fin_tc(a0, a1, dis, W, b):
    return pl.pallas_call(
        _fin_body,
        grid=(_GRID,),
        in_specs=[
            pl.BlockSpec((_RB, D), lambda i: (i, 0)),
            pl.BlockSpec((_RB, D), lambda i: (i, 0)),
            pl.BlockSpec((_RB, 1), lambda i: (i, 0)),
            pl.BlockSpec((D, D), lambda i: (0, 0)),
            pl.BlockSpec((1, D), lambda i: (0, 0)),
        ],
        out_specs=pl.BlockSpec((_RB, D), lambda i: (i, 0)),
        out_shape=jax.ShapeDtypeStruct((N, D), jnp.float32),
    )(a0, a1, dis, W, b)


def kernel(x, edge_index, W, b):
    row = edge_index[0]
    col = edge_index[1]
    pad = E_PAD - E
    row_p = jnp.concatenate([row, jnp.zeros((pad,), jnp.int32)])
    col_p = jnp.concatenate([col, jnp.full((pad,), N, jnp.int32)])
    row3 = row_p.reshape(NC, NS, N_CH, C)
    col3 = col_p.reshape(NC, NS, N_CH, C)

    deg_part = _deg_call(col3)
    p0 = deg_part[0, :N].reshape(N, 1)
    p1 = deg_part[1, :N].reshape(N, 1)
    xs, dis, dis2 = _prep_tc(p0, p1, x)

    a = _hop_call(xs, row3, col3)
    h1s = _mid_tc(a[0, :N], a[1, :N], dis2)
    q = _hop_call(h1s, row3, col3)
    out = _fin_tc(q[0, :N], q[1, :N], dis, W, b.reshape(1, D))
    return out


# speedup vs baseline: 13.9408x; 13.9408x over previous
import functools

import jax
import jax.numpy as jnp
from jax import lax
from jax.experimental import pallas as pl
from jax.experimental.pallas import tpu as pltpu
from jax.experimental.pallas import tpu_sc as plsc

NC = 2
NS = 16
NW = NC * NS
LANES = 16
C = 128

N = 10000
D = 128
DH = D // 2
E = 320000

N_CH_DEG = 2 * (-(-E // (NW * C * 2)))
E_PAD_DEG = NW * C * N_CH_DEG
DEG_PAD = 10240
DEG_PER_TILE = DEG_PAD // NS

N_CH = 2 * (-(-E // (NS * C * 2)))
E_PAD = NS * C * N_CH
ACC_ROWS = 10112
ROWS_PER_TILE = ACC_ROWS // NS


def _zero_vmem_2d(buf, rows, width):
    def body(r, _):
        z = jnp.zeros((LANES,), jnp.float32)
        for i in range(width // LANES):
            buf[r, pl.ds(i * LANES, LANES)] = z
        return 0
    lax.fori_loop(0, rows, body, 0)


def _zero_vmem_1d(buf, n):
    def body(r, _):
        buf[pl.ds(r * LANES, LANES)] = jnp.zeros((LANES,), jnp.float32)
        return 0
    lax.fori_loop(0, n // LANES, body, 0)


def _deg_body(col_hbm, out_hbm, col_v, ones_v, zrow_v, acc_s):
    c = lax.axis_index("c")
    s = lax.axis_index("s")
    pltpu.sync_copy(col_hbm.at[c, s], col_v)
    _zero_vmem_1d(zrow_v, DEG_PER_TILE)
    def wone(i, _):
        ones_v[pl.ds(i * LANES, LANES)] = jnp.full((LANES,), 1.0, jnp.float32)
        return 0
    lax.fori_loop(0, C // LANES, wone, 0)
    pltpu.sync_copy(zrow_v, acc_s.at[pl.ds(s * DEG_PER_TILE, DEG_PER_TILE)])
    plsc.subcore_barrier()
    def chunk(j, _):
        pltpu.sync_copy(ones_v, acc_s.at[col_v.at[j]], add=True)
        return 0
    lax.fori_loop(0, N_CH_DEG, chunk, 0)
    plsc.subcore_barrier()
    sl = pl.ds(s * DEG_PER_TILE, DEG_PER_TILE)
    pltpu.sync_copy(acc_s.at[sl], zrow_v)
    pltpu.sync_copy(zrow_v, out_hbm.at[c, sl])


def _hop_body(h0_hbm, h1_hbm, row_hbm, col_hbm, out_hbm,
              row_v, col_v, buf0, buf1, acc_s, gsem, ssem):
    c = lax.axis_index("c")
    s = lax.axis_index("s")
    pltpu.sync_copy(row_hbm.at[s], row_v)
    pltpu.sync_copy(col_hbm.at[s], col_v)
    _zero_vmem_2d(buf0, C, DH)
    base = s * ROWS_PER_TILE
    for k in range(4):
        pltpu.sync_copy(buf0, acc_s.at[pl.ds(base + k * C, C)])
    rem = ROWS_PER_TILE - 4 * C
    pltpu.sync_copy(buf0.at[pl.ds(0, rem)],
                    acc_s.at[pl.ds(base + 4 * C, rem)])
    plsc.subcore_barrier()

    half = N_CH // 2

    def gather_start(j, buf):
        @pl.when(c == 0)
        def _():
            pltpu.make_async_copy(h0_hbm.at[row_v.at[j]], buf, gsem).start()

        @pl.when(c == 1)
        def _():
            pltpu.make_async_copy(h1_hbm.at[row_v.at[j]], buf, gsem).start()

    def gather_wait(buf):
        pltpu.make_async_copy(h0_hbm.at[row_v.at[0]], buf, gsem).wait()

    def scat(j, buf):
        return pltpu.make_async_copy(buf, acc_s.at[col_v.at[j]], ssem)

    gather_start(0, buf0)

    def pair(p, _):
        j0 = 2 * p
        gather_wait(buf0)

        @pl.when(p > 0)
        def _():
            scat(j0, buf1).wait()
        gather_start(j0 + 1, buf1)
        scat(j0, buf0).start(add=True)
        gather_wait(buf1)

        @pl.when(p < half - 1)
        def _():
            gather_start(j0 + 2, buf0)
        scat(j0, buf0).wait()
        scat(j0 + 1, buf1).start(add=True)
        return 0

    lax.fori_loop(0, half, pair, 0)
    scat(0, buf1).wait()
    plsc.subcore_barrier()

    for k in range(4):
        sl = pl.ds(base + k * C, C)
        pltpu.sync_copy(acc_s.at[sl], buf0)
        pltpu.sync_copy(buf0, out_hbm.at[c, sl, :])
    sl = pl.ds(base + 4 * C, rem)
    pltpu.sync_copy(acc_s.at[sl], buf0.at[pl.ds(0, rem)])
    pltpu.sync_copy(buf0.at[pl.ds(0, rem)], out_hbm.at[c, sl, :])


_MESH = plsc.VectorSubcoreMesh(core_axis_name="c", subcore_axis_name="s",
                               num_cores=NC, num_subcores=NS)

_deg_call = pl.kernel(
    _deg_body,
    out_type=jax.ShapeDtypeStruct((NC, DEG_PAD), jnp.float32),
    mesh=_MESH,
    scratch_types=[
        pltpu.VMEM((N_CH_DEG, C), jnp.int32),
        pltpu.VMEM((C,), jnp.float32),
        pltpu.VMEM((DEG_PER_TILE,), jnp.float32),
        pltpu.VMEM_SHARED((DEG_PAD,), jnp.float32),
    ],
)

_hop_call = pl.kernel(
    _hop_body,
    out_type=jax.ShapeDtypeStruct((NC, ACC_ROWS, DH), jnp.float32),
    mesh=_MESH,
    compiler_params=pltpu.CompilerParams(use_tc_tiling_on_sc=False),
    scratch_types=[
        pltpu.VMEM((N_CH, C), jnp.int32),
        pltpu.VMEM((N_CH, C), jnp.int32),
        pltpu.VMEM((C, DH), jnp.float32),
        pltpu.VMEM((C, DH), jnp.float32),
        pltpu.VMEM_SHARED((ACC_ROWS, DH), jnp.float32),
        pltpu.SemaphoreType.DMA,
        pltpu.SemaphoreType.DMA,
    ],
)


_RB = 2000
_GRID = N // _RB


def _prep_body(p0, p1, x_ref, xs0, xs1, dis, dis2):
    deg = p0[...] + p1[...]
    d = lax.rsqrt(jnp.maximum(deg, 1e-12))
    dd = jnp.where(deg > 0, d, 0.0)
    dis[...] = dd
    dis2[...] = dd * dd
    xst = x_ref[...] * dd
    xs0[...] = xst[:, :DH]
    xs1[...] = xst[:, DH:]


def _prep_tc(p0, p1, x):
    return pl.pallas_call(
        _prep_body,
        grid=(_GRID,),
        in_specs=[
            pl.BlockSpec((_RB, 1), lambda i: (i, 0)),
            pl.BlockSpec((_RB, 1), lambda i: (i, 0)),
            pl.BlockSpec((_RB, D), lambda i: (i, 0)),
        ],
        out_specs=[
            pl.BlockSpec((_RB, DH), lambda i: (i, 0)),
            pl.BlockSpec((_RB, DH), lambda i: (i, 0)),
            pl.BlockSpec((_RB, 1), lambda i: (i, 0)),
            pl.BlockSpec((_RB, 1), lambda i: (i, 0)),
        ],
        out_shape=[
            jax.ShapeDtypeStruct((N, DH), jnp.float32),
            jax.ShapeDtypeStruct((N, DH), jnp.float32),
            jax.ShapeDtypeStruct((N, 1), jnp.float32),
            jax.ShapeDtypeStruct((N, 1), jnp.float32),
        ],
    )(p0, p1, x)


def _mid_body(a0, a1, dis2, y0, y1):
    y0[...] = a0[...] * dis2[...]
    y1[...] = a1[...] * dis2[...]


def _mid_tc(a0, a1, dis2):
    return pl.pallas_call(
        _mid_body,
        grid=(_GRID,),
        in_specs=[
            pl.BlockSpec((_RB, DH), lambda i: (i, 0)),
            pl.BlockSpec((_RB, DH), lambda i: (i, 0)),
            pl.BlockSpec((_RB, 1), lambda i: (i, 0)),
        ],
        out_specs=[
            pl.BlockSpec((_RB, DH), lambda i: (i, 0)),
            pl.BlockSpec((_RB, DH), lambda i: (i, 0)),
        ],
        out_shape=[
            jax.ShapeDtypeStruct((N, DH), jnp.float32),
            jax.ShapeDtypeStruct((N, DH), jnp.float32),
        ],
    )(a0, a1, dis2)


def _fin_body(a0, a1, dis, w_ref, b_ref, o):
    h0 = a0[...] * dis[...]
    h1 = a1[...] * dis[...]
    w = w_ref[...]
    o[...] = (
        lax.dot_general(h0, w[:, :DH], (((1,), (1,)), ((), ())),
                        preferred_element_type=jnp.float32)
        + lax.dot_general(h1, w[:, DH:], (((1,), (1,)), ((), ())),
                          preferred_element_type=jnp.float32)
        + b_ref[...]
    )


def _fin_tc(a0, a1, dis, W, b):
    return pl.pallas_call(
        _fin_body,
        grid=(_GRID,),
        in_specs=[
            pl.BlockSpec((_RB, DH), lambda i: (i, 0)),
            pl.BlockSpec((_RB, DH), lambda i: (i, 0)),
            pl.BlockSpec((_RB, 1), lambda i: (i, 0)),
            pl.BlockSpec((D, D), lambda i: (0, 0)),
            pl.BlockSpec((1, D), lambda i: (0, 0)),
        ],
        out_specs=pl.BlockSpec((_RB, D), lambda i: (i, 0)),
        out_shape=jax.ShapeDtypeStruct((N, D), jnp.float32),
    )(a0, a1, dis, W, b)


def kernel(x, edge_index, W, b):
    row = edge_index[0]
    col = edge_index[1]
    pad_d = E_PAD_DEG - E
    col_deg = jnp.concatenate([col, jnp.full((pad_d,), N, jnp.int32)])
    col_deg = col_deg.reshape(NC, NS, N_CH_DEG, C)

    pad_h = E_PAD - E
    row_h = jnp.concatenate([row, jnp.zeros((pad_h,), jnp.int32)])
    col_h = jnp.concatenate([col, jnp.full((pad_h,), N, jnp.int32)])
    row_h = row_h.reshape(NS, N_CH, C)
    col_h = col_h.reshape(NS, N_CH, C)

    deg_part = _deg_call(col_deg)
    p0 = deg_part[0, :N].reshape(N, 1)
    p1 = deg_part[1, :N].reshape(N, 1)
    xs0, xs1, dis, dis2 = _prep_tc(p0, p1, x)

    a = _hop_call(xs0, xs1, row_h, col_h)
    y0, y1 = _mid_tc(a[0, :N], a[1, :N], dis2)
    q = _hop_call(y0, y1, row_h, col_h)
    out = _fin_tc(q[0, :N], q[1, :N], dis, W, b.reshape(1, D))
    return out
